# baseline (speedup 1.0000x reference)
import functools

import jax
import jax.numpy as jnp
from jax import lax
from jax.experimental import pallas as pl
from jax.experimental.pallas import tpu as pltpu
from jax.experimental.pallas import tpu_sc as plsc

NUM_CLASSES = 1000000
EMBED_DIM = 32
BATCH = 16384
FIELDS = 26

_B = BATCH * FIELDS
_NC = 2
_NS = 16
_NW = _NC * _NS
_PER_W = _B // _NW
_CHUNK = 128
_NCHUNK = _PER_W // _CHUNK
_NBUF = 4
assert _PER_W * _NW == _B and _NCHUNK * _CHUNK == _PER_W
assert _NCHUNK % _NBUF == 0 and _CHUNK % 8 == 0


def _gather_body(idx_hbm, table_hbm, out_hbm, idx_v, *bufs):
    rows = bufs[:_NBUF]
    gsem = bufs[_NBUF:2 * _NBUF]
    ssem = bufs[2 * _NBUF:]

    wid = lax.axis_index("s") * _NC + lax.axis_index("c")
    base = wid * _PER_W
    pltpu.sync_copy(idx_hbm.at[pl.ds(base, _PER_W)], idx_v)

    def g_start(i, b):
        pltpu.make_async_copy(
            table_hbm.at[idx_v.at[pl.ds(i * _CHUNK, _CHUNK)]], rows[b], gsem[b]
        ).start()

    def g_wait(b):
        pltpu.make_async_copy(
            table_hbm.at[idx_v.at[pl.ds(0, _CHUNK)]], rows[b], gsem[b]
        ).wait()

    def s_start(i, b):
        pltpu.make_async_copy(
            rows[b], out_hbm.at[pl.ds(base + i * _CHUNK, _CHUNK)], ssem[b]
        ).start()

    def s_wait(b):
        pltpu.make_async_copy(
            rows[b], out_hbm.at[pl.ds(base, _CHUNK)], ssem[b]
        ).wait()

    for b in range(_NBUF):
        g_start(b, b)

    def body(gi, _):
        i0 = gi * _NBUF
        for b in range(_NBUF):
            i = i0 + b
            g_wait(b)
            s_start(i, b)
            s_wait(b)
            g_start(i + _NBUF, b)
        return 0

    lax.fori_loop(0, _NCHUNK // _NBUF - 1, body, 0)

    for b in range(_NBUF):
        g_wait(b)
        s_start(_NCHUNK - _NBUF + b, b)
        s_wait(b)


@jax.jit
def _embed_lookup(idx_flat, table):
    mesh = plsc.VectorSubcoreMesh(core_axis_name="c", subcore_axis_name="s")
    grab = pl.kernel(
        _gather_body,
        out_type=jax.ShapeDtypeStruct((_B, EMBED_DIM), jnp.float32),
        mesh=mesh,
        scratch_types=(
            [pltpu.VMEM((_PER_W,), jnp.int32)]
            + [pltpu.VMEM((_CHUNK, EMBED_DIM), jnp.float32)] * _NBUF
            + [pltpu.SemaphoreType.DMA] * (2 * _NBUF)
        ),
        compiler_params=pltpu.CompilerParams(use_tc_tiling_on_sc=False),
    )
    return grab(idx_flat, table)


def kernel(severity_ids, table):
    idx_flat = severity_ids.reshape(_B).astype(jnp.int32)
    out = _embed_lookup(idx_flat, table)
    return out.reshape(BATCH, FIELDS, EMBED_DIM)


# speedup vs baseline: 1.0013x; 1.0013x over previous
import functools

import jax
import jax.numpy as jnp
from jax import lax
from jax.experimental import pallas as pl
from jax.experimental.pallas import tpu as pltpu
from jax.experimental.pallas import tpu_sc as plsc

NUM_CLASSES = 1000000
EMBED_DIM = 32
BATCH = 16384
FIELDS = 26

_B = BATCH * FIELDS
_NC = 2
_NS = 16
_NW = _NC * _NS
_PER_W = _B // _NW
_CHUNK = 128
_NCHUNK = _PER_W // _CHUNK
_NBUF = 4
assert _PER_W * _NW == _B and _NCHUNK * _CHUNK == _PER_W
assert _NCHUNK % _NBUF == 0 and _CHUNK % 8 == 0


def _gather_body(idx_hbm, table_hbm, out_hbm, idx_v, *bufs):
    rows = bufs[:_NBUF]
    gsem = bufs[_NBUF:2 * _NBUF]
    ssem = bufs[2 * _NBUF:]

    wid = lax.axis_index("s") * _NC + lax.axis_index("c")
    base = wid * _PER_W
    pltpu.sync_copy(idx_hbm.at[pl.ds(base, _PER_W)], idx_v)

    def g_start(i, b):
        for j in range(_CHUNK // 16):
            iv = idx_v[pl.ds(i * _CHUNK + j * 16, 16)]
            pltpu.make_async_copy(
                table_hbm.at[iv], rows[b].at[pl.ds(j * 16, 16)], gsem[b]
            ).start()

    def g_wait(b):
        pltpu.make_async_copy(
            table_hbm.at[idx_v.at[pl.ds(0, _CHUNK)]], rows[b], gsem[b]
        ).wait()

    def s_start(i, b):
        pltpu.make_async_copy(
            rows[b], out_hbm.at[pl.ds(base + i * _CHUNK, _CHUNK)], ssem[b]
        ).start()

    def s_wait(b):
        pltpu.make_async_copy(
            rows[b], out_hbm.at[pl.ds(base, _CHUNK)], ssem[b]
        ).wait()

    for b in range(_NBUF):
        g_start(b, b)

    def body(gi, _):
        i0 = gi * _NBUF
        for b in range(_NBUF):
            i = i0 + b
            g_wait(b)
            s_start(i, b)
            s_wait(b)
            g_start(i + _NBUF, b)
        return 0

    lax.fori_loop(0, _NCHUNK // _NBUF - 1, body, 0)

    for b in range(_NBUF):
        g_wait(b)
        s_start(_NCHUNK - _NBUF + b, b)
        s_wait(b)


@jax.jit
def _embed_lookup(idx_flat, table):
    mesh = plsc.VectorSubcoreMesh(core_axis_name="c", subcore_axis_name="s")
    grab = pl.kernel(
        _gather_body,
        out_type=jax.ShapeDtypeStruct((_B, EMBED_DIM), jnp.float32),
        mesh=mesh,
        scratch_types=(
            [pltpu.VMEM((_PER_W,), jnp.int32)]
            + [pltpu.VMEM((_CHUNK, EMBED_DIM), jnp.float32)] * _NBUF
            + [pltpu.SemaphoreType.DMA] * (2 * _NBUF)
        ),
        compiler_params=pltpu.CompilerParams(use_tc_tiling_on_sc=False),
    )
    return grab(idx_flat, table)


def kernel(severity_ids, table):
    idx_flat = severity_ids.reshape(_B).astype(jnp.int32)
    out = _embed_lookup(idx_flat, table)
    return out.reshape(BATCH, FIELDS, EMBED_DIM)
